# baseline (speedup 1.0000x reference)
import functools

import jax
import jax.numpy as jnp
from jax import lax
from jax.experimental import pallas as pl
from jax.experimental.pallas import tpu as pltpu
from jax.experimental.pallas import tpu_sc as plsc

HIDDEN = 4096
NC, NS = 2, 16
NW = NC * NS
K = 4
NBUF = 6
AHEAD = NBUF - 1


@functools.partial(jax.jit, static_argnames=("batch",))
def _embedding_lookup(ids, table, *, batch):
    b_per_w = batch // NW
    nsteps = b_per_w // K
    mesh = plsc.VectorSubcoreMesh(
        core_axis_name="c", subcore_axis_name="s", num_cores=NC, num_subcores=NS
    )

    @functools.partial(
        pl.kernel,
        out_type=jax.ShapeDtypeStruct((batch, HIDDEN), jnp.float32),
        mesh=mesh,
        scratch_types=[
            pltpu.VMEM((nsteps, K), jnp.int32),
            [pltpu.VMEM((K, HIDDEN), jnp.float32) for _ in range(NBUF)],
            [pltpu.SemaphoreType.DMA for _ in range(NBUF)],
            [pltpu.SemaphoreType.DMA for _ in range(NBUF)],
        ],
    )
    def body(table_hbm, ids_hbm, out_hbm, idx_v, rows, gsem, ssem):
        wid = lax.axis_index("s") * NC + lax.axis_index("c")
        base = wid * b_per_w
        pltpu.sync_copy(ids_hbm.at[wid], idx_v)

        def g_copy(s, b):
            return pltpu.make_async_copy(table_hbm.at[idx_v.at[s]], rows[b], gsem[b])

        def s_copy(s, b):
            dst = out_hbm.at[pl.ds(base + s * K, K)]
            return pltpu.make_async_copy(rows[b], dst, ssem[b])

        for s in range(AHEAD):
            g_copy(s, s % NBUF).start()

        def consume(s, b):
            g_copy(s, b).wait()
            s_copy(s, b).start()

        def issue(s, sn, bn, wait_store):
            if wait_store:
                s_copy(sn - NBUF, bn).wait()
            g_copy(sn, bn).start()

        head = NBUF - AHEAD
        for s in range(head):
            consume(s, s % NBUF)
            issue(s, s + AHEAD, (s + AHEAD) % NBUF, wait_store=False)

        lo, hi = head, nsteps - AHEAD
        n_loop = ((hi - lo) // NBUF) * NBUF

        @pl.loop(lo, lo + n_loop, step=NBUF)
        def _(i):
            for j in range(NBUF):
                s = i + j
                b = (lo + j) % NBUF
                consume(s, b)
                issue(s, s + AHEAD, (b + AHEAD) % NBUF, wait_store=True)

        for s in range(lo + n_loop, hi):
            consume(s, s % NBUF)
            issue(s, s + AHEAD, (s + AHEAD) % NBUF, wait_store=True)

        for s in range(hi, nsteps):
            consume(s, s % NBUF)
        for s in range(nsteps - NBUF, nsteps):
            s_copy(s, s % NBUF).wait()

    return body(table, ids)


TC_ROWS = 8


@functools.partial(jax.jit, static_argnames=("batch",))
def _embedding_lookup_tc(ids, table, *, batch):
    nblocks = batch // TC_ROWS

    def body(idx_ref, *refs):
        out_ref = refs[TC_ROWS]
        for j in range(TC_ROWS):
            out_ref[pl.ds(j, 1), :] = refs[j][0]

    grid_spec = pltpu.PrefetchScalarGridSpec(
        num_scalar_prefetch=1,
        grid=(nblocks,),
        in_specs=[
            pl.BlockSpec(
                (1, 1, HIDDEN),
                functools.partial(lambda j, i, idx: (idx[i * TC_ROWS + j], 0, 0), j),
            )
            for j in range(TC_ROWS)
        ],
        out_specs=pl.BlockSpec((TC_ROWS, HIDDEN), lambda i, idx: (i, 0)),
    )
    table3 = table.reshape(table.shape[0], 1, HIDDEN)
    return pl.pallas_call(
        body,
        grid_spec=grid_spec,
        out_shape=jax.ShapeDtypeStruct((batch, HIDDEN), jnp.float32),
    )(ids, *([table3] * TC_ROWS))


def kernel(input_ids, embed_tokens):
    batch = input_ids.size
    b_sc = (batch * 5 // 8) // 256 * 256
    ids_flat = input_ids.reshape(-1).astype(jnp.int32)
    ids_sc = ids_flat[:b_sc].reshape(NW, b_sc // (NW * K), K)
    out_sc = _embedding_lookup(ids_sc, embed_tokens, batch=b_sc)
    out_tc = _embedding_lookup_tc(ids_flat[b_sc:], embed_tokens, batch=batch - b_sc)
    out = jnp.concatenate([out_sc, out_tc], axis=0)
    return out.reshape(*input_ids.shape, HIDDEN)


# speedup vs baseline: 2.6821x; 2.6821x over previous
import functools

import jax
import jax.numpy as jnp
from jax import lax
from jax.experimental import pallas as pl
from jax.experimental.pallas import tpu as pltpu
from jax.experimental.pallas import tpu_sc as plsc

HIDDEN = 4096
NC, NS = 2, 16
NW = NC * NS
K = 4
NBUF = 6
AHEAD = NBUF - 1


@functools.partial(jax.jit, static_argnames=("batch",))
def _embedding_lookup(ids, table, *, batch):
    b_per_w = batch // NW
    nsteps = b_per_w // K
    mesh = plsc.VectorSubcoreMesh(
        core_axis_name="c", subcore_axis_name="s", num_cores=NC, num_subcores=NS
    )

    @functools.partial(
        pl.kernel,
        out_type=jax.ShapeDtypeStruct((batch, HIDDEN), jnp.float32),
        mesh=mesh,
        scratch_types=[
            pltpu.VMEM((nsteps, K), jnp.int32),
            [pltpu.VMEM((K, HIDDEN), jnp.float32) for _ in range(NBUF)],
            [pltpu.SemaphoreType.DMA for _ in range(NBUF)],
            [pltpu.SemaphoreType.DMA for _ in range(NBUF)],
        ],
    )
    def body(table_hbm, ids_hbm, out_hbm, idx_v, rows, gsem, ssem):
        wid = lax.axis_index("s") * NC + lax.axis_index("c")
        base = wid * b_per_w
        pltpu.sync_copy(ids_hbm.at[wid], idx_v)

        def g_copy(s, b):
            return pltpu.make_async_copy(table_hbm.at[idx_v.at[s]], rows[b], gsem[b])

        def s_copy(s, b):
            dst = out_hbm.at[pl.ds(base + s * K, K)]
            return pltpu.make_async_copy(rows[b], dst, ssem[b])

        for s in range(AHEAD):
            g_copy(s, s % NBUF).start()

        def consume(s, b):
            g_copy(s, b).wait()
            s_copy(s, b).start()

        def issue(s, sn, bn, wait_store):
            if wait_store:
                s_copy(sn - NBUF, bn).wait()
            g_copy(sn, bn).start()

        head = NBUF - AHEAD
        for s in range(head):
            consume(s, s % NBUF)
            issue(s, s + AHEAD, (s + AHEAD) % NBUF, wait_store=False)

        lo, hi = head, nsteps - AHEAD
        n_loop = ((hi - lo) // NBUF) * NBUF

        @pl.loop(lo, lo + n_loop, step=NBUF)
        def _(i):
            for j in range(NBUF):
                s = i + j
                b = (lo + j) % NBUF
                consume(s, b)
                issue(s, s + AHEAD, (b + AHEAD) % NBUF, wait_store=True)

        for s in range(lo + n_loop, hi):
            consume(s, s % NBUF)
            issue(s, s + AHEAD, (s + AHEAD) % NBUF, wait_store=True)

        for s in range(hi, nsteps):
            consume(s, s % NBUF)
        for s in range(nsteps - NBUF, nsteps):
            s_copy(s, s % NBUF).wait()

    return body(table, ids)


TC_ROWS = 8


@functools.partial(jax.jit, static_argnames=("batch",))
def _embedding_lookup_tc(ids, table, *, batch):
    nblocks = batch // TC_ROWS

    def body(idx_ref, table_ref, out_ref, buf, insem):
        i = pl.program_id(0)

        def fetch(chunk, slot):
            for j in range(TC_ROWS):
                pltpu.make_async_copy(
                    table_ref.at[pl.ds(idx_ref[chunk * TC_ROWS + j], 1)],
                    buf.at[slot, pl.ds(j, 1)],
                    insem.at[slot],
                ).start()

        @pl.when(i == 0)
        def _():
            fetch(0, 0)

        @pl.when(i + 1 < nblocks)
        def _():
            fetch(i + 1, (i + 1) % 2)

        for j in range(TC_ROWS):
            pltpu.make_async_copy(
                table_ref.at[pl.ds(0, 1)], buf.at[i % 2, pl.ds(j, 1)], insem.at[i % 2]
            ).wait()
        out_ref[...] = buf[i % 2]

    grid_spec = pltpu.PrefetchScalarGridSpec(
        num_scalar_prefetch=1,
        grid=(nblocks,),
        in_specs=[pl.BlockSpec(memory_space=pltpu.HBM)],
        out_specs=pl.BlockSpec((TC_ROWS, HIDDEN), lambda i, idx: (i, 0)),
        scratch_shapes=[
            pltpu.VMEM((2, TC_ROWS, HIDDEN), jnp.float32),
            pltpu.SemaphoreType.DMA((2,)),
        ],
    )
    return pl.pallas_call(
        body,
        grid_spec=grid_spec,
        out_shape=jax.ShapeDtypeStruct((batch, HIDDEN), jnp.float32),
    )(ids, table)


def kernel(input_ids, embed_tokens):
    batch = input_ids.size
    b_sc = (batch * 5 // 8) // 256 * 256
    ids_flat = input_ids.reshape(-1).astype(jnp.int32)
    ids_sc = ids_flat[:b_sc].reshape(NW, b_sc // (NW * K), K)
    out_sc = _embedding_lookup(ids_sc, embed_tokens, batch=b_sc)
    out_tc = _embedding_lookup_tc(ids_flat[b_sc:], embed_tokens, batch=batch - b_sc)
    out = jnp.concatenate([out_sc, out_tc], axis=0)
    return out.reshape(*input_ids.shape, HIDDEN)


# speedup vs baseline: 8.2402x; 3.0722x over previous
import functools

import jax
import jax.numpy as jnp
from jax import lax
from jax.experimental import pallas as pl
from jax.experimental.pallas import tpu as pltpu
from jax.experimental.pallas import tpu_sc as plsc

HIDDEN = 4096
NC, NS = 2, 16
NW = NC * NS
K = 4
NBUF = 6
AHEAD = NBUF - 1


@functools.partial(jax.jit, static_argnames=("batch",))
def _embedding_lookup(ids, table, *, batch):
    b_per_w = batch // NW
    nsteps = b_per_w // K
    mesh = plsc.VectorSubcoreMesh(
        core_axis_name="c", subcore_axis_name="s", num_cores=NC, num_subcores=NS
    )

    @functools.partial(
        pl.kernel,
        out_type=jax.ShapeDtypeStruct((batch, HIDDEN), jnp.float32),
        mesh=mesh,
        scratch_types=[
            pltpu.VMEM((nsteps, K), jnp.int32),
            [pltpu.VMEM((K, HIDDEN), jnp.float32) for _ in range(NBUF)],
            [pltpu.SemaphoreType.DMA for _ in range(NBUF)],
            [pltpu.SemaphoreType.DMA for _ in range(NBUF)],
        ],
    )
    def body(table_hbm, ids_hbm, out_hbm, idx_v, rows, gsem, ssem):
        wid = lax.axis_index("s") * NC + lax.axis_index("c")
        base = wid * b_per_w
        pltpu.sync_copy(ids_hbm.at[wid], idx_v)

        def g_copy(s, b):
            return pltpu.make_async_copy(table_hbm.at[idx_v.at[s]], rows[b], gsem[b])

        def s_copy(s, b):
            dst = out_hbm.at[pl.ds(base + s * K, K)]
            return pltpu.make_async_copy(rows[b], dst, ssem[b])

        for s in range(AHEAD):
            g_copy(s, s % NBUF).start()

        def consume(s, b):
            g_copy(s, b).wait()
            s_copy(s, b).start()

        def issue(s, sn, bn, wait_store):
            if wait_store:
                s_copy(sn - NBUF, bn).wait()
            g_copy(sn, bn).start()

        head = NBUF - AHEAD
        for s in range(head):
            consume(s, s % NBUF)
            issue(s, s + AHEAD, (s + AHEAD) % NBUF, wait_store=False)

        lo, hi = head, nsteps - AHEAD
        n_loop = ((hi - lo) // NBUF) * NBUF

        @pl.loop(lo, lo + n_loop, step=NBUF)
        def _(i):
            for j in range(NBUF):
                s = i + j
                b = (lo + j) % NBUF
                consume(s, b)
                issue(s, s + AHEAD, (b + AHEAD) % NBUF, wait_store=True)

        for s in range(lo + n_loop, hi):
            consume(s, s % NBUF)
            issue(s, s + AHEAD, (s + AHEAD) % NBUF, wait_store=True)

        for s in range(hi, nsteps):
            consume(s, s % NBUF)
        for s in range(nsteps - NBUF, nsteps):
            s_copy(s, s % NBUF).wait()

    return body(table, ids)


def kernel(input_ids, embed_tokens):
    batch = input_ids.size
    ids = input_ids.reshape(NW, batch // (NW * K), K).astype(jnp.int32)
    out = _embedding_lookup(ids, embed_tokens, batch=batch)
    return out.reshape(*input_ids.shape, HIDDEN)
